# baseline (speedup 1.0000x reference)
import jax
import jax.numpy as jnp
from jax import lax
from jax.experimental import pallas as pl
from jax.experimental.pallas import tpu as pltpu
from jax.experimental.pallas import tpu_sc as plsc

NN = 10000
NP = 10240
NE = 640000
NEP = 655360
DF = 100
FF = 128
FH = 64
NG = 64
CUT = 12.0

NC = 2
NS = 16
NW = NC * NS
BEP = 800
CE = 1280
BE = 160
NBK = CE // BE
NCH = NEP // NC // NS // CE
NB = 1024
EB = 2048

_mesh = plsc.VectorSubcoreMesh(core_axis_name="c", subcore_axis_name="s")
_sc_params = pltpu.CompilerParams(needs_layout_passes=False)


def _sc_prep_body(z_hbm, emb_hbm, px_hbm, py_hbm, pz_hbm, row_hbm, col_hbm,
                  h_out, sq_out,
                  zidx_v, rows_v, px_v, py_v, pz_v, ridx_v, cidx_v, sq_v, sem):
    c = lax.axis_index("c")
    s = lax.axis_index("s")
    wid = s * NC + c
    nb = NP // NW
    base = wid * nb
    pltpu.sync_copy(z_hbm.at[pl.ds(base, nb)], zidx_v)
    pltpu.async_copy(emb_hbm.at[zidx_v], rows_v, sem).wait()
    pltpu.sync_copy(rows_v, h_out.at[pl.ds(base, nb)])
    pltpu.sync_copy(px_hbm, px_v)
    pltpu.sync_copy(py_hbm, py_v)
    pltpu.sync_copy(pz_hbm, pz_v)
    et = NE // NW
    ebase = wid * et

    def blk(b, carry):
        off = ebase + b * BEP
        pltpu.sync_copy(row_hbm.at[pl.ds(off, BEP)], ridx_v)
        pltpu.sync_copy(col_hbm.at[pl.ds(off, BEP)], cidx_v)

        def inner(i, carry2):
            ri = ridx_v[pl.ds(i * 16, 16)]
            ci = cidx_v[pl.ds(i * 16, 16)]
            dx = plsc.load_gather(px_v, [ri]) - plsc.load_gather(px_v, [ci])
            dy = plsc.load_gather(py_v, [ri]) - plsc.load_gather(py_v, [ci])
            dz = plsc.load_gather(pz_v, [ri]) - plsc.load_gather(pz_v, [ci])
            sq_v[pl.ds(i * 16, 16)] = dx * dx + dy * dy + dz * dz + 1e-12
            return carry2

        lax.fori_loop(0, BEP // 16, inner, 0)
        pltpu.sync_copy(sq_v, sq_out.at[pl.ds(off, BEP)])
        return carry

    lax.fori_loop(0, et // BEP, blk, 0)


_sc_prep = pl.kernel(
    _sc_prep_body,
    out_type=[jax.ShapeDtypeStruct((NP, FF), jnp.float32),
              jax.ShapeDtypeStruct((NE,), jnp.float32)],
    mesh=_mesh,
    scratch_types=[
        pltpu.VMEM((NP // NW,), jnp.int32),
        pltpu.VMEM((NP // NW, FF), jnp.float32),
        pltpu.VMEM((NN,), jnp.float32),
        pltpu.VMEM((NN,), jnp.float32),
        pltpu.VMEM((NN,), jnp.float32),
        pltpu.VMEM((BEP,), jnp.int32),
        pltpu.VMEM((BEP,), jnp.int32),
        pltpu.VMEM((BEP,), jnp.float32),
        pltpu.SemaphoreType.DMA,
    ],
    compiler_params=_sc_params,
)


def _sc_edge_body(row2_hbm, col2_hbm, g_hbm, y_hbm, zeros_hbm, m_out,
                  ridx_v, cidx_v, rblk_v, cblk_v, g_v, y_v, m_sh, sg, sy):
    c = lax.axis_index("c")
    s = lax.axis_index("s")

    rs = NP // NS
    pltpu.sync_copy(zeros_hbm.at[pl.ds(s * rs, rs)],
                    m_sh.at[pl.ds(s * rs, rs)])
    plsc.subcore_barrier()

    et = NEP // NC // NS
    tbase = c * (NEP // NC) + s * et

    def chunk(ch, carry):
        eoff = tbase + ch * CE
        crow = c * (NEP // NC // BE) + s * (et // BE) + ch * NBK
        pltpu.sync_copy(row2_hbm.at[pl.ds(crow, NBK)], ridx_v)
        pltpu.sync_copy(col2_hbm.at[pl.ds(crow, NBK)], cidx_v)

        for b in range(NBK):
            @plsc.parallel_loop(0, BE // 16, unroll=5)
            def cpidx(i):
                rblk_v[pl.ds(i * 16, 16)] = ridx_v[b, pl.ds(i * 16, 16)]
                cblk_v[pl.ds(i * 16, 16)] = cidx_v[b, pl.ds(i * 16, 16)]

            gd = pltpu.async_copy(g_hbm.at[pl.ds(eoff + b * BE, BE)], g_v, sg)
            yd = pltpu.async_copy(y_hbm.at[rblk_v], y_v, sy)
            gd.wait()
            yd.wait()

            @plsc.parallel_loop(0, BE, unroll=8)
            def mulrow(r):
                for k in range(FF // 16):
                    y_v[r, pl.ds(k * 16, 16)] = (
                        y_v[r, pl.ds(k * 16, 16)] * g_v[r, pl.ds(k * 16, 16)])

            pltpu.sync_copy(y_v, m_sh.at[cblk_v], add=True)
        return carry

    lax.fori_loop(0, NCH, chunk, 0)
    plsc.subcore_barrier()
    pltpu.sync_copy(m_sh.at[pl.ds(s * rs, rs)],
                    m_out.at[pl.ds(c * NP + s * rs, rs)])


_sc_edge = pl.kernel(
    _sc_edge_body,
    out_type=[jax.ShapeDtypeStruct((NC * NP, FF), jnp.float32)],
    mesh=_mesh,
    scratch_types=[
        pltpu.VMEM((NBK, BE), jnp.int32),
        pltpu.VMEM((NBK, BE), jnp.int32),
        pltpu.VMEM((BE,), jnp.int32),
        pltpu.VMEM((BE,), jnp.int32),
        pltpu.VMEM((BE, FF), jnp.float32),
        pltpu.VMEM((BE, FF), jnp.float32),
        pltpu.VMEM_SHARED((NP, FF), jnp.float32),
        pltpu.SemaphoreType.DMA,
        pltpu.SemaphoreType.DMA,
    ],
    compiler_params=_sc_params,
)


def _tc_edge_g_body(sq_ref, cen_ref, wid_ref, w1_ref, b1_ref, w2_ref, b2_ref,
                    g1_ref, g2_ref):
    sq = sq_ref[...]
    d = jnp.sqrt(sq)
    x = d * (1.0 / CUT)
    x3 = x * x * x
    x4 = x3 * x
    x5 = x4 * x
    cutf = jnp.where(x < 1.0, 1.0 - 6.0 * x5 + 15.0 * x4 - 10.0 * x3,
                     jnp.zeros_like(x))
    t = jnp.exp(-d)
    diff = t - cen_ref[...]
    ea = cutf * jnp.exp(-wid_ref[...] * diff * diff)
    g1_ref[...] = (jnp.dot(ea, w1_ref[...], preferred_element_type=jnp.float32)
                   + b1_ref[...])
    g2_ref[...] = (jnp.dot(ea, w2_ref[...], preferred_element_type=jnp.float32)
                   + b2_ref[...])


def _tc_edge_g(sq2, cen, wd, w1, b1, w2, b2):
    grid = (NEP // EB,)
    return pl.pallas_call(
        _tc_edge_g_body,
        grid=grid,
        in_specs=[
            pl.BlockSpec((EB, 1), lambda i: (i, 0)),
            pl.BlockSpec((1, DF), lambda i: (0, 0)),
            pl.BlockSpec((1, DF), lambda i: (0, 0)),
            pl.BlockSpec((DF, FF), lambda i: (0, 0)),
            pl.BlockSpec((1, FF), lambda i: (0, 0)),
            pl.BlockSpec((DF, FF), lambda i: (0, 0)),
            pl.BlockSpec((1, FF), lambda i: (0, 0)),
        ],
        out_specs=[pl.BlockSpec((EB, FF), lambda i: (i, 0)),
                   pl.BlockSpec((EB, FF), lambda i: (i, 0))],
        out_shape=[jax.ShapeDtypeStruct((NEP, FF), jnp.float32),
                   jax.ShapeDtypeStruct((NEP, FF), jnp.float32)],
    )(sq2, cen, wd, w1, b1, w2, b2)


def _tc_node_a_body(h_ref, w_ref, b_ref, y_ref):
    y_ref[...] = (jnp.dot(jnp.maximum(h_ref[...], 0.0), w_ref[...],
                          preferred_element_type=jnp.float32) + b_ref[...])


def _tc_node_a(h, w, b):
    return pl.pallas_call(
        _tc_node_a_body,
        grid=(NP // NB,),
        in_specs=[
            pl.BlockSpec((NB, FF), lambda i: (i, 0)),
            pl.BlockSpec((FF, FF), lambda i: (0, 0)),
            pl.BlockSpec((1, FF), lambda i: (0, 0)),
        ],
        out_specs=pl.BlockSpec((NB, FF), lambda i: (i, 0)),
        out_shape=jax.ShapeDtypeStruct((NP, FF), jnp.float32),
    )(h, w, b)


def _tc_node_b_body(m0_ref, m1_ref, h_ref, wm_ref, wb_ref, u_ref, wo_ref,
                    bo_ref, os_ref, h_out, y_out, os_out):
    wm = wm_ref[...]
    wb = wb_ref[...]

    def dot(x, w):
        return jnp.dot(x, w, preferred_element_type=jnp.float32)

    def res(x, k):
        hh = jnp.maximum(x, 0.0)
        return x + dot(dot(hh, wm[k]) + wb[k], wm[k + 1]) + wb[k + 1]

    m = m0_ref[0] + m1_ref[0]
    m = res(m, 0)
    m = res(m, 2)
    m = jnp.maximum(m, 0.0)
    hn = u_ref[...] * h_ref[...] + dot(m, wm[4]) + wb[4]
    hn = res(hn, 5)
    hn = res(hn, 7)
    ho = res(hn, 9)
    ho = res(ho, 11)
    ho = jnp.maximum(ho, 0.0)
    os_out[...] = os_ref[...] + dot(ho, wo_ref[...]) + bo_ref[...]
    y_out[...] = dot(jnp.maximum(hn, 0.0), wm[13]) + wb[13]
    h_out[...] = hn


def _tc_node_b(m2, h, wm, wb, u, wo, bo, os_in):
    return pl.pallas_call(
        _tc_node_b_body,
        grid=(NP // NB,),
        in_specs=[
            pl.BlockSpec((1, NB, FF), lambda i: (0, i, 0)),
            pl.BlockSpec((1, NB, FF), lambda i: (1, i, 0)),
            pl.BlockSpec((NB, FF), lambda i: (i, 0)),
            pl.BlockSpec((14, FF, FF), lambda i: (0, 0, 0)),
            pl.BlockSpec((14, FF), lambda i: (0, 0)),
            pl.BlockSpec((1, FF), lambda i: (0, 0)),
            pl.BlockSpec((FF, 1), lambda i: (0, 0)),
            pl.BlockSpec((1, 1), lambda i: (0, 0)),
            pl.BlockSpec((NB, 1), lambda i: (i, 0)),
        ],
        out_specs=[pl.BlockSpec((NB, FF), lambda i: (i, 0)),
                   pl.BlockSpec((NB, FF), lambda i: (i, 0)),
                   pl.BlockSpec((NB, 1), lambda i: (i, 0))],
        out_shape=[jax.ShapeDtypeStruct((NP, FF), jnp.float32),
                   jax.ShapeDtypeStruct((NP, FF), jnp.float32),
                   jax.ShapeDtypeStruct((NP, 1), jnp.float32)],
    )(m2, m2, h, wm, wb, u, wo, bo, os_in)


def _tc_energy_body(bf_ref, os_ref, e_ref):
    i = pl.program_id(0)

    @pl.when(i == 0)
    def _init():
        e_ref[...] = jnp.zeros_like(e_ref)

    iota = lax.broadcasted_iota(jnp.int32, (1, NG), 1).astype(jnp.float32)
    onehot = (bf_ref[...] == iota).astype(jnp.float32)
    e_ref[...] += lax.dot_general(onehot, os_ref[...],
                                  (((0,), (0,)), ((), ())),
                                  preferred_element_type=jnp.float32)


def _tc_energy(bf, os):
    return pl.pallas_call(
        _tc_energy_body,
        grid=(NP // NB,),
        in_specs=[pl.BlockSpec((NB, 1), lambda i: (i, 0)),
                  pl.BlockSpec((NB, 1), lambda i: (i, 0))],
        out_specs=pl.BlockSpec((NG, 1), lambda i: (0, 0)),
        out_shape=jax.ShapeDtypeStruct((NG, 1), jnp.float32),
    )(bf, os)


def _stack_weights(blk, outblk, wi_next, bi_next):
    mats, vecs = [], []
    for r in blk["res"]:
        mats += [r["dense"]["W"], r["residual"]["W"]]
        vecs += [r["dense"]["b"], r["residual"]["b"]]
    mats.append(blk["dense"]["W"])
    vecs.append(blk["dense"]["b"])
    for r in blk["atomic_res"]:
        mats += [r["dense"]["W"], r["residual"]["W"]]
        vecs += [r["dense"]["b"], r["residual"]["b"]]
    for r in outblk["res"]:
        mats += [r["dense"]["W"], r["residual"]["W"]]
        vecs += [r["dense"]["b"], r["residual"]["b"]]
    mats.append(wi_next)
    vecs.append(bi_next)
    return jnp.stack(mats), jnp.stack(vecs)


def kernel(z, pos, edge_index, batch, emb, rbf_centers, rbf_widths,
           interactions, outputs):
    f32 = jnp.float32
    row = edge_index[0].astype(jnp.int32)
    col = edge_index[1].astype(jnp.int32)
    row_p = jnp.concatenate([row, jnp.zeros((NEP - NE,), jnp.int32)])
    col_p = jnp.concatenate([col, jnp.full((NEP - NE,), NP - 1, jnp.int32)])
    row2 = row_p.reshape(NEP // BE, BE)
    col2 = col_p.reshape(NEP // BE, BE)
    z_pad = jnp.concatenate([z.astype(jnp.int32),
                             jnp.zeros((NP - NN,), jnp.int32)])
    posx = pos[:, 0]
    posy = pos[:, 1]
    posz = pos[:, 2]
    batch_f = jnp.concatenate([batch.astype(f32),
                               jnp.full((NP - NN,), float(NG), f32)])
    batch_f = batch_f.reshape(NP, 1)

    h, sq = _sc_prep(z_pad, emb, posx, posy, posz, row, col)
    sq2 = jnp.concatenate([sq, jnp.zeros((NEP - NE,), f32)]).reshape(NEP, 1)
    cen = rbf_centers.reshape(1, DF)
    wd = rbf_widths.reshape(1, DF)
    i1, i2 = interactions
    g1, g2 = _tc_edge_g(sq2, cen, wd,
                        i1["d2f"]["W"], i1["d2f"]["b"].reshape(1, FF),
                        i2["d2f"]["W"], i2["d2f"]["b"].reshape(1, FF))

    y1 = _tc_node_a(h, i1["dense_i"]["W"], i1["dense_i"]["b"].reshape(1, FF))

    zeros_m = jnp.zeros((NP, FF), f32)
    os0 = jnp.zeros((NP, 1), f32)

    wm1, wb1 = _stack_weights(i1, outputs[0], i2["dense_i"]["W"],
                              i2["dense_i"]["b"])
    wm2, wb2 = _stack_weights(i2, outputs[1], i2["dense_i"]["W"],
                              i2["dense_i"]["b"])

    (m1,) = _sc_edge(row2, col2, g1, y1, zeros_m)
    m1 = m1.reshape(NC, NP, FF)
    h2, y2, os1 = _tc_node_b(m1, h, wm1, wb1, i1["u"].reshape(1, FF),
                             outputs[0]["dense"]["W"],
                             outputs[0]["dense"]["b"].reshape(1, 1), os0)
    (m2,) = _sc_edge(row2, col2, g2, y2, zeros_m)
    m2 = m2.reshape(NC, NP, FF)
    _h3, _y3, os2 = _tc_node_b(m2, h2, wm2, wb2, i2["u"].reshape(1, FF),
                               outputs[1]["dense"]["W"],
                               outputs[1]["dense"]["b"].reshape(1, 1), os1)
    return _tc_energy(batch_f, os2)


# speedup vs baseline: 1.6396x; 1.6396x over previous
import jax
import jax.numpy as jnp
from jax import lax
from jax.experimental import pallas as pl
from jax.experimental.pallas import tpu as pltpu
from jax.experimental.pallas import tpu_sc as plsc

NN = 10000
NP = 10240
NE = 640000
NEP = 655360
DF = 100
FF = 128
FH = 64
NG = 64
CUT = 12.0

NC = 2
NS = 16
NW = NC * NS
BEP = 800
CE = 1280
BE = 160
NBK = CE // BE
NCH = NEP // NC // NS // CE
NB = 1024
EB = 2048

_mesh = plsc.VectorSubcoreMesh(core_axis_name="c", subcore_axis_name="s")
_sc_params = pltpu.CompilerParams(needs_layout_passes=False)


def _sc_prep_body(z_hbm, emb_hbm, px_hbm, py_hbm, pz_hbm, row_hbm, col_hbm,
                  h_out, sq_out,
                  zidx_v, rows_v, px_v, py_v, pz_v, ridx_v, cidx_v, sq_v, sem):
    c = lax.axis_index("c")
    s = lax.axis_index("s")
    wid = s * NC + c
    nb = NP // NW
    base = wid * nb
    pltpu.sync_copy(z_hbm.at[pl.ds(base, nb)], zidx_v)
    pltpu.async_copy(emb_hbm.at[zidx_v], rows_v, sem).wait()
    pltpu.sync_copy(rows_v, h_out.at[pl.ds(base, nb)])
    pltpu.sync_copy(px_hbm, px_v)
    pltpu.sync_copy(py_hbm, py_v)
    pltpu.sync_copy(pz_hbm, pz_v)
    et = NE // NW
    ebase = wid * et

    def blk(b, carry):
        off = ebase + b * BEP
        pltpu.sync_copy(row_hbm.at[pl.ds(off, BEP)], ridx_v)
        pltpu.sync_copy(col_hbm.at[pl.ds(off, BEP)], cidx_v)

        def inner(i, carry2):
            ri = ridx_v[pl.ds(i * 16, 16)]
            ci = cidx_v[pl.ds(i * 16, 16)]
            dx = plsc.load_gather(px_v, [ri]) - plsc.load_gather(px_v, [ci])
            dy = plsc.load_gather(py_v, [ri]) - plsc.load_gather(py_v, [ci])
            dz = plsc.load_gather(pz_v, [ri]) - plsc.load_gather(pz_v, [ci])
            sq_v[pl.ds(i * 16, 16)] = dx * dx + dy * dy + dz * dz + 1e-12
            return carry2

        lax.fori_loop(0, BEP // 16, inner, 0)
        pltpu.sync_copy(sq_v, sq_out.at[pl.ds(off, BEP)])
        return carry

    lax.fori_loop(0, et // BEP, blk, 0)


_sc_prep = pl.kernel(
    _sc_prep_body,
    out_type=[jax.ShapeDtypeStruct((NP, FF), jnp.float32),
              jax.ShapeDtypeStruct((NE,), jnp.float32)],
    mesh=_mesh,
    scratch_types=[
        pltpu.VMEM((NP // NW,), jnp.int32),
        pltpu.VMEM((NP // NW, FF), jnp.float32),
        pltpu.VMEM((NN,), jnp.float32),
        pltpu.VMEM((NN,), jnp.float32),
        pltpu.VMEM((NN,), jnp.float32),
        pltpu.VMEM((BEP,), jnp.int32),
        pltpu.VMEM((BEP,), jnp.int32),
        pltpu.VMEM((BEP,), jnp.float32),
        pltpu.SemaphoreType.DMA,
    ],
    compiler_params=_sc_params,
)


def _sc_edge_body(row2_hbm, col2_hbm, g_hbm, y_hbm, zeros_hbm, m_out,
                  ridx_v, cidx_v, rblk_v, cblk_v, g_v, y_v, m_sh, sg, sy):
    c = lax.axis_index("c")
    s = lax.axis_index("s")

    rs = NP // NS
    pltpu.sync_copy(zeros_hbm.at[pl.ds(s * rs, rs)],
                    m_sh.at[pl.ds(s * rs, rs)])
    plsc.subcore_barrier()

    et = NEP // NC // NS
    tbase = c * (NEP // NC) + s * et

    def chunk(ch, carry):
        eoff = tbase + ch * CE
        crow = c * (NEP // NC // BE) + s * (et // BE) + ch * NBK
        pltpu.sync_copy(row2_hbm.at[pl.ds(crow, NBK)], ridx_v)
        pltpu.sync_copy(col2_hbm.at[pl.ds(crow, NBK)], cidx_v)

        for b in range(NBK):
            @plsc.parallel_loop(0, BE // 16, unroll=5)
            def cpidx(i):
                rblk_v[pl.ds(i * 16, 16)] = ridx_v[b, pl.ds(i * 16, 16)]
                cblk_v[pl.ds(i * 16, 16)] = cidx_v[b, pl.ds(i * 16, 16)]

            gd = pltpu.async_copy(g_hbm.at[pl.ds(eoff + b * BE, BE)], g_v, sg)
            yd = pltpu.async_copy(y_hbm.at[rblk_v], y_v, sy)
            gd.wait()
            yd.wait()

            @plsc.parallel_loop(0, BE, unroll=8)
            def mulrow(r):
                for k in range(FF // 16):
                    y_v[r, pl.ds(k * 16, 16)] = (
                        y_v[r, pl.ds(k * 16, 16)] * g_v[r, pl.ds(k * 16, 16)])

            pltpu.sync_copy(y_v, m_sh.at[cblk_v], add=True)
        return carry

    lax.fori_loop(0, NCH, chunk, 0)
    plsc.subcore_barrier()
    pltpu.sync_copy(m_sh.at[pl.ds(s * rs, rs)],
                    m_out.at[pl.ds(c * NP + s * rs, rs)])


_sc_edge = pl.kernel(
    _sc_edge_body,
    out_type=[jax.ShapeDtypeStruct((NC * NP, FF), jnp.float32)],
    mesh=_mesh,
    scratch_types=[
        pltpu.VMEM((NBK, BE), jnp.int32),
        pltpu.VMEM((NBK, BE), jnp.int32),
        pltpu.VMEM((BE,), jnp.int32),
        pltpu.VMEM((BE,), jnp.int32),
        pltpu.VMEM((BE, FF), jnp.float32),
        pltpu.VMEM((BE, FF), jnp.float32),
        pltpu.VMEM_SHARED((NP, FF), jnp.float32),
        pltpu.SemaphoreType.DMA,
        pltpu.SemaphoreType.DMA,
    ],
    compiler_params=_sc_params,
)


def _tc_edge_g_body(sq_ref, cen_ref, wid_ref, w1_ref, b1_ref, w2_ref, b2_ref,
                    g1_ref, g2_ref):
    sq = sq_ref[...]
    d = jnp.sqrt(sq)
    x = d * (1.0 / CUT)
    x3 = x * x * x
    x4 = x3 * x
    x5 = x4 * x
    cutf = jnp.where(x < 1.0, 1.0 - 6.0 * x5 + 15.0 * x4 - 10.0 * x3,
                     jnp.zeros_like(x))
    t = jnp.exp(-d)
    diff = t - cen_ref[...]
    ea = cutf * jnp.exp(-wid_ref[...] * diff * diff)
    g1_ref[...] = (jnp.dot(ea, w1_ref[...], preferred_element_type=jnp.float32)
                   + b1_ref[...])
    g2_ref[...] = (jnp.dot(ea, w2_ref[...], preferred_element_type=jnp.float32)
                   + b2_ref[...])


def _tc_edge_g(sq2, cen, wd, w1, b1, w2, b2):
    grid = (NEP // EB,)
    return pl.pallas_call(
        _tc_edge_g_body,
        grid=grid,
        in_specs=[
            pl.BlockSpec((EB, 1), lambda i: (i, 0)),
            pl.BlockSpec((1, DF), lambda i: (0, 0)),
            pl.BlockSpec((1, DF), lambda i: (0, 0)),
            pl.BlockSpec((DF, FF), lambda i: (0, 0)),
            pl.BlockSpec((1, FF), lambda i: (0, 0)),
            pl.BlockSpec((DF, FF), lambda i: (0, 0)),
            pl.BlockSpec((1, FF), lambda i: (0, 0)),
        ],
        out_specs=[pl.BlockSpec((EB, FF), lambda i: (i, 0)),
                   pl.BlockSpec((EB, FF), lambda i: (i, 0))],
        out_shape=[jax.ShapeDtypeStruct((NEP, FF), jnp.float32),
                   jax.ShapeDtypeStruct((NEP, FF), jnp.float32)],
    )(sq2, cen, wd, w1, b1, w2, b2)


def _tc_node_a_body(h_ref, w_ref, b_ref, y_ref):
    y_ref[...] = (jnp.dot(jnp.maximum(h_ref[...], 0.0), w_ref[...],
                          preferred_element_type=jnp.float32) + b_ref[...])


def _tc_node_a(h, w, b):
    return pl.pallas_call(
        _tc_node_a_body,
        grid=(NP // NB,),
        in_specs=[
            pl.BlockSpec((NB, FF), lambda i: (i, 0)),
            pl.BlockSpec((FF, FF), lambda i: (0, 0)),
            pl.BlockSpec((1, FF), lambda i: (0, 0)),
        ],
        out_specs=pl.BlockSpec((NB, FF), lambda i: (i, 0)),
        out_shape=jax.ShapeDtypeStruct((NP, FF), jnp.float32),
    )(h, w, b)


def _tc_node_b_body(m0_ref, m1_ref, h_ref, wm_ref, wb_ref, u_ref, wo_ref,
                    bo_ref, os_ref, h_out, y_out, os_out):
    wm = wm_ref[...]
    wb = wb_ref[...]

    def dot(x, w):
        return jnp.dot(x, w, preferred_element_type=jnp.float32)

    def res(x, k):
        hh = jnp.maximum(x, 0.0)
        return x + dot(dot(hh, wm[k]) + wb[k], wm[k + 1]) + wb[k + 1]

    m = m0_ref[0] + m1_ref[0]
    m = res(m, 0)
    m = res(m, 2)
    m = jnp.maximum(m, 0.0)
    hn = u_ref[...] * h_ref[...] + dot(m, wm[4]) + wb[4]
    hn = res(hn, 5)
    hn = res(hn, 7)
    ho = res(hn, 9)
    ho = res(ho, 11)
    ho = jnp.maximum(ho, 0.0)
    os_out[...] = os_ref[...] + dot(ho, wo_ref[...]) + bo_ref[...]
    y_out[...] = dot(jnp.maximum(hn, 0.0), wm[13]) + wb[13]
    h_out[...] = hn


def _tc_node_b(m2, h, wm, wb, u, wo, bo, os_in):
    return pl.pallas_call(
        _tc_node_b_body,
        grid=(NP // NB,),
        in_specs=[
            pl.BlockSpec((1, NB, FF), lambda i: (0, i, 0)),
            pl.BlockSpec((1, NB, FF), lambda i: (1, i, 0)),
            pl.BlockSpec((NB, FF), lambda i: (i, 0)),
            pl.BlockSpec((14, FF, FF), lambda i: (0, 0, 0)),
            pl.BlockSpec((14, FF), lambda i: (0, 0)),
            pl.BlockSpec((1, FF), lambda i: (0, 0)),
            pl.BlockSpec((FF, 1), lambda i: (0, 0)),
            pl.BlockSpec((1, 1), lambda i: (0, 0)),
            pl.BlockSpec((NB, 1), lambda i: (i, 0)),
        ],
        out_specs=[pl.BlockSpec((NB, FF), lambda i: (i, 0)),
                   pl.BlockSpec((NB, FF), lambda i: (i, 0)),
                   pl.BlockSpec((NB, 1), lambda i: (i, 0))],
        out_shape=[jax.ShapeDtypeStruct((NP, FF), jnp.float32),
                   jax.ShapeDtypeStruct((NP, FF), jnp.float32),
                   jax.ShapeDtypeStruct((NP, 1), jnp.float32)],
    )(m2, m2, h, wm, wb, u, wo, bo, os_in)


def _tc_energy_body(bf_ref, os_ref, e_ref):
    i = pl.program_id(0)

    @pl.when(i == 0)
    def _init():
        e_ref[...] = jnp.zeros_like(e_ref)

    iota = lax.broadcasted_iota(jnp.int32, (1, NG), 1).astype(jnp.float32)
    onehot = (bf_ref[...] == iota).astype(jnp.float32)
    e_ref[...] += lax.dot_general(onehot, os_ref[...],
                                  (((0,), (0,)), ((), ())),
                                  preferred_element_type=jnp.float32)


def _tc_energy(bf, os):
    return pl.pallas_call(
        _tc_energy_body,
        grid=(NP // NB,),
        in_specs=[pl.BlockSpec((NB, 1), lambda i: (i, 0)),
                  pl.BlockSpec((NB, 1), lambda i: (i, 0))],
        out_specs=pl.BlockSpec((NG, 1), lambda i: (0, 0)),
        out_shape=jax.ShapeDtypeStruct((NG, 1), jnp.float32),
    )(bf, os)


def _stack_weights(blk, outblk, wi_next, bi_next):
    mats, vecs = [], []
    for r in blk["res"]:
        mats += [r["dense"]["W"], r["residual"]["W"]]
        vecs += [r["dense"]["b"], r["residual"]["b"]]
    mats.append(blk["dense"]["W"])
    vecs.append(blk["dense"]["b"])
    for r in blk["atomic_res"]:
        mats += [r["dense"]["W"], r["residual"]["W"]]
        vecs += [r["dense"]["b"], r["residual"]["b"]]
    for r in outblk["res"]:
        mats += [r["dense"]["W"], r["residual"]["W"]]
        vecs += [r["dense"]["b"], r["residual"]["b"]]
    mats.append(wi_next)
    vecs.append(bi_next)
    return jnp.stack(mats), jnp.stack(vecs)


def kernel(z, pos, edge_index, batch, emb, rbf_centers, rbf_widths,
           interactions, outputs):
    f32 = jnp.float32
    row = edge_index[0].astype(jnp.int32)
    col = edge_index[1].astype(jnp.int32)
    pad_i = jnp.arange(NEP - NE, dtype=jnp.int32)
    row_p = jnp.concatenate([row, pad_i % NN])
    col_p = jnp.concatenate([col, NN + pad_i % (NP - NN)])
    row2 = row_p.reshape(NEP // BE, BE)
    col2 = col_p.reshape(NEP // BE, BE)
    z_pad = jnp.concatenate([z.astype(jnp.int32),
                             jnp.zeros((NP - NN,), jnp.int32)])
    posx = pos[:, 0]
    posy = pos[:, 1]
    posz = pos[:, 2]
    batch_f = jnp.concatenate([batch.astype(f32),
                               jnp.full((NP - NN,), float(NG), f32)])
    batch_f = batch_f.reshape(NP, 1)

    h, sq = _sc_prep(z_pad, emb, posx, posy, posz, row, col)
    sq2 = jnp.concatenate([sq, jnp.zeros((NEP - NE,), f32)]).reshape(NEP, 1)
    cen = rbf_centers.reshape(1, DF)
    wd = rbf_widths.reshape(1, DF)
    i1, i2 = interactions
    g1, g2 = _tc_edge_g(sq2, cen, wd,
                        i1["d2f"]["W"], i1["d2f"]["b"].reshape(1, FF),
                        i2["d2f"]["W"], i2["d2f"]["b"].reshape(1, FF))

    y1 = _tc_node_a(h, i1["dense_i"]["W"], i1["dense_i"]["b"].reshape(1, FF))

    zeros_m = jnp.zeros((NP, FF), f32)
    os0 = jnp.zeros((NP, 1), f32)

    wm1, wb1 = _stack_weights(i1, outputs[0], i2["dense_i"]["W"],
                              i2["dense_i"]["b"])
    wm2, wb2 = _stack_weights(i2, outputs[1], i2["dense_i"]["W"],
                              i2["dense_i"]["b"])

    (m1,) = _sc_edge(row2, col2, g1, y1, zeros_m)
    m1 = m1.reshape(NC, NP, FF)
    h2, y2, os1 = _tc_node_b(m1, h, wm1, wb1, i1["u"].reshape(1, FF),
                             outputs[0]["dense"]["W"],
                             outputs[0]["dense"]["b"].reshape(1, 1), os0)
    (m2,) = _sc_edge(row2, col2, g2, y2, zeros_m)
    m2 = m2.reshape(NC, NP, FF)
    _h3, _y3, os2 = _tc_node_b(m2, h2, wm2, wb2, i2["u"].reshape(1, FF),
                               outputs[1]["dense"]["W"],
                               outputs[1]["dense"]["b"].reshape(1, 1), os1)
    return _tc_energy(batch_f, os2)


# speedup vs baseline: 1.6450x; 1.0033x over previous
import jax
import jax.numpy as jnp
from jax import lax
from jax.experimental import pallas as pl
from jax.experimental.pallas import tpu as pltpu
from jax.experimental.pallas import tpu_sc as plsc

NN = 10000
NP = 10240
NE = 640000
NEP = 655360
DF = 100
FF = 128
FH = 64
NG = 64
CUT = 12.0

NC = 2
NS = 16
NW = NC * NS
BEP = 800
CE = 1280
BE = 160
NBK = CE // BE
NCH = NEP // NC // NS // CE
NB = 1024
EB = 2048

_mesh = plsc.VectorSubcoreMesh(core_axis_name="c", subcore_axis_name="s")
_sc_params = pltpu.CompilerParams(needs_layout_passes=False)


def _sc_prep_body(z_hbm, emb_hbm, px_hbm, py_hbm, pz_hbm, row_hbm, col_hbm,
                  h_out, sq_out,
                  zidx_v, rows_v, px_v, py_v, pz_v, ridx_v, cidx_v, sq_v, sem):
    c = lax.axis_index("c")
    s = lax.axis_index("s")
    wid = s * NC + c
    nb = NP // NW
    base = wid * nb
    pltpu.sync_copy(z_hbm.at[pl.ds(base, nb)], zidx_v)
    pltpu.async_copy(emb_hbm.at[zidx_v], rows_v, sem).wait()
    pltpu.sync_copy(rows_v, h_out.at[pl.ds(base, nb)])
    pltpu.sync_copy(px_hbm, px_v)
    pltpu.sync_copy(py_hbm, py_v)
    pltpu.sync_copy(pz_hbm, pz_v)
    et = NE // NW
    ebase = wid * et

    def blk(b, carry):
        off = ebase + b * BEP
        pltpu.sync_copy(row_hbm.at[pl.ds(off, BEP)], ridx_v)
        pltpu.sync_copy(col_hbm.at[pl.ds(off, BEP)], cidx_v)

        def inner(i, carry2):
            ri = ridx_v[pl.ds(i * 16, 16)]
            ci = cidx_v[pl.ds(i * 16, 16)]
            dx = plsc.load_gather(px_v, [ri]) - plsc.load_gather(px_v, [ci])
            dy = plsc.load_gather(py_v, [ri]) - plsc.load_gather(py_v, [ci])
            dz = plsc.load_gather(pz_v, [ri]) - plsc.load_gather(pz_v, [ci])
            sq_v[pl.ds(i * 16, 16)] = dx * dx + dy * dy + dz * dz + 1e-12
            return carry2

        lax.fori_loop(0, BEP // 16, inner, 0)
        pltpu.sync_copy(sq_v, sq_out.at[pl.ds(off, BEP)])
        return carry

    lax.fori_loop(0, et // BEP, blk, 0)


_sc_prep = pl.kernel(
    _sc_prep_body,
    out_type=[jax.ShapeDtypeStruct((NP, FF), jnp.float32),
              jax.ShapeDtypeStruct((NE,), jnp.float32)],
    mesh=_mesh,
    scratch_types=[
        pltpu.VMEM((NP // NW,), jnp.int32),
        pltpu.VMEM((NP // NW, FF), jnp.float32),
        pltpu.VMEM((NN,), jnp.float32),
        pltpu.VMEM((NN,), jnp.float32),
        pltpu.VMEM((NN,), jnp.float32),
        pltpu.VMEM((BEP,), jnp.int32),
        pltpu.VMEM((BEP,), jnp.int32),
        pltpu.VMEM((BEP,), jnp.float32),
        pltpu.SemaphoreType.DMA,
    ],
    compiler_params=_sc_params,
)


def _sc_edge_body(row2_hbm, col2_hbm, g_hbm, y_hbm, zeros_hbm, m_out,
                  ridx_v, cidx_v, rblk_v, cblk_v, g_v, y_v, m_sh, sg, sy):
    c = lax.axis_index("c")
    s = lax.axis_index("s")

    rs = NP // NS
    pltpu.sync_copy(zeros_hbm.at[pl.ds(s * rs, rs)],
                    m_sh.at[pl.ds(s * rs, rs)])
    plsc.subcore_barrier()

    et = NEP // NC // NS
    tbase = c * (NEP // NC) + s * et

    def chunk(ch, carry):
        eoff = tbase + ch * CE
        crow = c * (NEP // NC // BE) + s * (et // BE) + ch * NBK
        pltpu.sync_copy(row2_hbm.at[pl.ds(crow, NBK)], ridx_v)
        pltpu.sync_copy(col2_hbm.at[pl.ds(crow, NBK)], cidx_v)

        for b in range(NBK):
            @plsc.parallel_loop(0, BE // 16, unroll=5)
            def cpidx(i):
                rblk_v[pl.ds(i * 16, 16)] = ridx_v[b, pl.ds(i * 16, 16)]
                cblk_v[pl.ds(i * 16, 16)] = cidx_v[b, pl.ds(i * 16, 16)]

            gd = pltpu.async_copy(g_hbm.at[pl.ds(eoff + b * BE, BE)], g_v, sg)
            yd = pltpu.async_copy(y_hbm.at[rblk_v], y_v, sy)
            gd.wait()
            yd.wait()

            @plsc.parallel_loop(0, BE, unroll=8)
            def mulrow(r):
                for k in range(FF // 16):
                    y_v[r, pl.ds(k * 16, 16)] = (
                        y_v[r, pl.ds(k * 16, 16)] * g_v[r, pl.ds(k * 16, 16)])

            pltpu.sync_copy(y_v, m_sh.at[cblk_v], add=True)
        return carry

    lax.fori_loop(0, NCH, chunk, 0)
    plsc.subcore_barrier()
    pltpu.sync_copy(m_sh.at[pl.ds(s * rs, rs)],
                    m_out.at[pl.ds(c * NP + s * rs, rs)])


_sc_edge = pl.kernel(
    _sc_edge_body,
    out_type=[jax.ShapeDtypeStruct((NC * NP, FF), jnp.float32)],
    mesh=_mesh,
    scratch_types=[
        pltpu.VMEM((NBK, BE), jnp.int32),
        pltpu.VMEM((NBK, BE), jnp.int32),
        pltpu.VMEM((BE,), jnp.int32),
        pltpu.VMEM((BE,), jnp.int32),
        pltpu.VMEM((BE, FF), jnp.float32),
        pltpu.VMEM((BE, FF), jnp.float32),
        pltpu.VMEM_SHARED((NP, FF), jnp.float32),
        pltpu.SemaphoreType.DMA,
        pltpu.SemaphoreType.DMA,
    ],
    compiler_params=_sc_params,
)


def _tc_edge_g_body(sq_ref, cen_ref, wid_ref, w1_ref, b1_ref, g1_ref):
    sq = sq_ref[...]
    d = jnp.sqrt(sq)
    x = d * (1.0 / CUT)
    x3 = x * x * x
    x4 = x3 * x
    x5 = x4 * x
    cutf = jnp.where(x < 1.0, 1.0 - 6.0 * x5 + 15.0 * x4 - 10.0 * x3,
                     jnp.zeros_like(x))
    t = jnp.exp(-d)
    diff = t - cen_ref[...]
    ea = cutf * jnp.exp(-wid_ref[...] * diff * diff)
    g1_ref[...] = (jnp.dot(ea, w1_ref[...], preferred_element_type=jnp.float32)
                   + b1_ref[...])


def _tc_edge_g(sq2, cen, wd, w1, b1):
    grid = (NEP // EB,)
    return pl.pallas_call(
        _tc_edge_g_body,
        grid=grid,
        in_specs=[
            pl.BlockSpec((EB, 1), lambda i: (i, 0)),
            pl.BlockSpec((1, DF), lambda i: (0, 0)),
            pl.BlockSpec((1, DF), lambda i: (0, 0)),
            pl.BlockSpec((DF, FF), lambda i: (0, 0)),
            pl.BlockSpec((1, FF), lambda i: (0, 0)),
        ],
        out_specs=pl.BlockSpec((EB, FF), lambda i: (i, 0)),
        out_shape=jax.ShapeDtypeStruct((NEP, FF), jnp.float32),
    )(sq2, cen, wd, w1, b1)


def _tc_node_a_body(h_ref, w_ref, b_ref, y_ref):
    y_ref[...] = (jnp.dot(jnp.maximum(h_ref[...], 0.0), w_ref[...],
                          preferred_element_type=jnp.float32) + b_ref[...])


def _tc_node_a(h, w, b):
    return pl.pallas_call(
        _tc_node_a_body,
        grid=(NP // NB,),
        in_specs=[
            pl.BlockSpec((NB, FF), lambda i: (i, 0)),
            pl.BlockSpec((FF, FF), lambda i: (0, 0)),
            pl.BlockSpec((1, FF), lambda i: (0, 0)),
        ],
        out_specs=pl.BlockSpec((NB, FF), lambda i: (i, 0)),
        out_shape=jax.ShapeDtypeStruct((NP, FF), jnp.float32),
    )(h, w, b)


def _tc_node_b_body(m0_ref, m1_ref, h_ref, wm_ref, wb_ref, u_ref, wo_ref,
                    bo_ref, os_ref, h_out, y_out, os_out):
    wm = wm_ref[...]
    wb = wb_ref[...]

    def dot(x, w):
        return jnp.dot(x, w, preferred_element_type=jnp.float32)

    def res(x, k):
        hh = jnp.maximum(x, 0.0)
        return x + dot(dot(hh, wm[k]) + wb[k], wm[k + 1]) + wb[k + 1]

    m = m0_ref[0] + m1_ref[0]
    m = res(m, 0)
    m = res(m, 2)
    m = jnp.maximum(m, 0.0)
    hn = u_ref[...] * h_ref[...] + dot(m, wm[4]) + wb[4]
    hn = res(hn, 5)
    hn = res(hn, 7)
    ho = res(hn, 9)
    ho = res(ho, 11)
    ho = jnp.maximum(ho, 0.0)
    os_out[...] = os_ref[...] + dot(ho, wo_ref[...]) + bo_ref[...]
    y_out[...] = dot(jnp.maximum(hn, 0.0), wm[13]) + wb[13]
    h_out[...] = hn


def _tc_node_b(m2, h, wm, wb, u, wo, bo, os_in):
    return pl.pallas_call(
        _tc_node_b_body,
        grid=(NP // NB,),
        in_specs=[
            pl.BlockSpec((1, NB, FF), lambda i: (0, i, 0)),
            pl.BlockSpec((1, NB, FF), lambda i: (1, i, 0)),
            pl.BlockSpec((NB, FF), lambda i: (i, 0)),
            pl.BlockSpec((14, FF, FF), lambda i: (0, 0, 0)),
            pl.BlockSpec((14, FF), lambda i: (0, 0)),
            pl.BlockSpec((1, FF), lambda i: (0, 0)),
            pl.BlockSpec((FF, 1), lambda i: (0, 0)),
            pl.BlockSpec((1, 1), lambda i: (0, 0)),
            pl.BlockSpec((NB, 1), lambda i: (i, 0)),
        ],
        out_specs=[pl.BlockSpec((NB, FF), lambda i: (i, 0)),
                   pl.BlockSpec((NB, FF), lambda i: (i, 0)),
                   pl.BlockSpec((NB, 1), lambda i: (i, 0))],
        out_shape=[jax.ShapeDtypeStruct((NP, FF), jnp.float32),
                   jax.ShapeDtypeStruct((NP, FF), jnp.float32),
                   jax.ShapeDtypeStruct((NP, 1), jnp.float32)],
    )(m2, m2, h, wm, wb, u, wo, bo, os_in)


def _tc_energy_body(bf_ref, os_ref, e_ref):
    i = pl.program_id(0)

    @pl.when(i == 0)
    def _init():
        e_ref[...] = jnp.zeros_like(e_ref)

    iota = lax.broadcasted_iota(jnp.int32, (1, NG), 1).astype(jnp.float32)
    onehot = (bf_ref[...] == iota).astype(jnp.float32)
    e_ref[...] += lax.dot_general(onehot, os_ref[...],
                                  (((0,), (0,)), ((), ())),
                                  preferred_element_type=jnp.float32)


def _tc_energy(bf, os):
    return pl.pallas_call(
        _tc_energy_body,
        grid=(NP // NB,),
        in_specs=[pl.BlockSpec((NB, 1), lambda i: (i, 0)),
                  pl.BlockSpec((NB, 1), lambda i: (i, 0))],
        out_specs=pl.BlockSpec((NG, 1), lambda i: (0, 0)),
        out_shape=jax.ShapeDtypeStruct((NG, 1), jnp.float32),
    )(bf, os)


def _stack_weights(blk, outblk, wi_next, bi_next):
    mats, vecs = [], []
    for r in blk["res"]:
        mats += [r["dense"]["W"], r["residual"]["W"]]
        vecs += [r["dense"]["b"], r["residual"]["b"]]
    mats.append(blk["dense"]["W"])
    vecs.append(blk["dense"]["b"])
    for r in blk["atomic_res"]:
        mats += [r["dense"]["W"], r["residual"]["W"]]
        vecs += [r["dense"]["b"], r["residual"]["b"]]
    for r in outblk["res"]:
        mats += [r["dense"]["W"], r["residual"]["W"]]
        vecs += [r["dense"]["b"], r["residual"]["b"]]
    mats.append(wi_next)
    vecs.append(bi_next)
    return jnp.stack(mats), jnp.stack(vecs)


def kernel(z, pos, edge_index, batch, emb, rbf_centers, rbf_widths,
           interactions, outputs):
    f32 = jnp.float32
    row = edge_index[0].astype(jnp.int32)
    col = edge_index[1].astype(jnp.int32)
    pad_i = jnp.arange(NEP - NE, dtype=jnp.int32)
    row_p = jnp.concatenate([row, pad_i % NN])
    col_p = jnp.concatenate([col, NN + pad_i % (NP - NN)])
    row2 = row_p.reshape(NEP // BE, BE)
    col2 = col_p.reshape(NEP // BE, BE)
    z_pad = jnp.concatenate([z.astype(jnp.int32),
                             jnp.zeros((NP - NN,), jnp.int32)])
    posx = pos[:, 0]
    posy = pos[:, 1]
    posz = pos[:, 2]
    batch_f = jnp.concatenate([batch.astype(f32),
                               jnp.full((NP - NN,), float(NG), f32)])
    batch_f = batch_f.reshape(NP, 1)

    h, sq = _sc_prep(z_pad, emb, posx, posy, posz, row, col)
    sq2 = jnp.concatenate([sq, jnp.zeros((NEP - NE,), f32)]).reshape(NEP, 1)
    cen = rbf_centers.reshape(1, DF)
    wd = rbf_widths.reshape(1, DF)
    i1, i2 = interactions
    g1 = _tc_edge_g(sq2, cen, wd, i1["d2f"]["W"], i1["d2f"]["b"].reshape(1, FF))
    g2 = _tc_edge_g(sq2, cen, wd, i2["d2f"]["W"], i2["d2f"]["b"].reshape(1, FF))

    y1 = _tc_node_a(h, i1["dense_i"]["W"], i1["dense_i"]["b"].reshape(1, FF))

    zeros_m = jnp.zeros((NP, FF), f32)
    os0 = jnp.zeros((NP, 1), f32)

    wm1, wb1 = _stack_weights(i1, outputs[0], i2["dense_i"]["W"],
                              i2["dense_i"]["b"])
    wm2, wb2 = _stack_weights(i2, outputs[1], i2["dense_i"]["W"],
                              i2["dense_i"]["b"])

    (m1,) = _sc_edge(row2, col2, g1, y1, zeros_m)
    m1 = m1.reshape(NC, NP, FF)
    h2, y2, os1 = _tc_node_b(m1, h, wm1, wb1, i1["u"].reshape(1, FF),
                             outputs[0]["dense"]["W"],
                             outputs[0]["dense"]["b"].reshape(1, 1), os0)
    (m2,) = _sc_edge(row2, col2, g2, y2, zeros_m)
    m2 = m2.reshape(NC, NP, FF)
    _h3, _y3, os2 = _tc_node_b(m2, h2, wm2, wb2, i2["u"].reshape(1, FF),
                               outputs[1]["dense"]["W"],
                               outputs[1]["dense"]["b"].reshape(1, 1), os1)
    return _tc_energy(batch_f, os2)
